# baseline (speedup 1.0000x reference)
import jax
import jax.numpy as jnp
from jax import lax
from jax.experimental import pallas as pl
from jax.experimental.pallas import tpu as pltpu

_B, _P, _D, _H = 16, 2048, 1024, 24
_D4 = _D // 4
_CHUNK = 256
_NCH = _P // _CHUNK
_NBUF = 8
_XDIM = _D + _H * _D4
_KCHUNK = 1024
_NK = _XDIM // _KCHUNK


def _body(n_ref, ne_ref, cb_ref, ci_ref, s_ref,
          ctx_ref, wp_ref, bp_ref, out_ref, abuf, asem, acc_ref):
    def chunk_copy(b, i, slot):
        return pltpu.make_async_copy(
            ctx_ref.at[b, pl.ds(i * _CHUNK, _CHUNK), :], abuf.at[slot],
            asem.at[slot])

    total = s_ref[0]

    def prime(j, c):
        chunk_copy(cb_ref[j], ci_ref[j], lax.rem(j, _NBUF)).start()
        return c

    lax.fori_loop(0, jnp.minimum(total, _NBUF), prime, 0)

    hvec = lax.broadcasted_iota(jnp.int32, (_H, 1), 0)

    def consume(j, c):
        slot = lax.rem(j, _NBUF)
        b = cb_ref[j]
        i = ci_ref[j]
        n = n_ref[b]
        bin_size = jnp.maximum(n // _H, 1)
        start = hvec * bin_size
        end = jnp.minimum(start + bin_size, n)
        chunk_copy(b, i, slot).wait()

        @pl.when(i == 0)
        def _():
            acc_ref[...] = jnp.zeros_like(acc_ref)

        pos = lax.broadcasted_iota(jnp.int32, (_H, _CHUNK), 1) + i * _CHUNK
        m = ((pos >= start) & (pos < end)).astype(jnp.bfloat16)
        acc_ref[...] += jnp.dot(m, abuf[slot].astype(jnp.bfloat16),
                                preferred_element_type=jnp.float32)

        @pl.when(j + _NBUF < total)
        def _():
            chunk_copy(cb_ref[j + _NBUF], ci_ref[j + _NBUF], slot).start()

        @pl.when(i + 1 == ne_ref[b])
        def _():
            inv = 1.0 / jnp.maximum((end - start).astype(jnp.float32), 1.0)
            nonempty = (start < n).astype(jnp.float32)
            seg_mean = acc_ref[...] * (inv * nonempty)
            hourly = lax.dot_general(
                seg_mean, wp_ref[...], (((1,), (1,)), ((), ())),
                preferred_element_type=jnp.float32)
            out_ref[pl.ds(b, 1)] = (hourly + bp_ref[...] * nonempty)[None]
        return c

    lax.fori_loop(0, total, consume, 0)


def _mlp_body(x_ref, w1_ref, b1_ref, w2_ref, b2_ref, out_ref, acc_ref):
    k = pl.program_id(0)

    @pl.when(k == 0)
    def _():
        acc_ref[...] = jnp.zeros_like(acc_ref)

    acc_ref[...] += lax.dot_general(
        x_ref[...], w1_ref[...], (((1,), (1,)), ((), ())),
        preferred_element_type=jnp.float32)

    @pl.when(k == _NK - 1)
    def _():
        y = acc_ref[...] + b1_ref[...]
        y = 0.5 * y * (1.0 + lax.erf(y * 0.7071067811865476))
        out_ref[...] = lax.dot_general(
            y, w2_ref[...], (((1,), (1,)), ((), ())),
            preferred_element_type=jnp.float32) + b2_ref[...]


def kernel(day_embed, episode_ctx, n_episodes, Wp, bp, W1, b1, W2, b2):
    n = n_episodes.astype(jnp.int32)
    bin_size = jnp.maximum(n // _H, 1)
    needed = jnp.minimum(_H * bin_size, n)
    nch = jnp.maximum((needed + _CHUNK - 1) // _CHUNK, 1)
    off = jnp.concatenate([jnp.zeros((1,), jnp.int32),
                           jnp.cumsum(nch, dtype=jnp.int32)])
    total = off[_B]
    j = jnp.arange(_B * _NCH, dtype=jnp.int32)
    cb = jnp.clip(jnp.searchsorted(off, j, side='right') - 1, 0, _B - 1)
    ci = j - off[cb]
    cb = cb.astype(jnp.int32)
    ci = ci.astype(jnp.int32)

    hourly = pl.pallas_call(
        _body,
        grid_spec=pltpu.PrefetchScalarGridSpec(
            num_scalar_prefetch=5,
            grid=(1,),
            in_specs=[
                pl.BlockSpec(memory_space=pl.ANY),
                pl.BlockSpec((_D4, _D), lambda i, *s: (0, 0)),
                pl.BlockSpec((1, _D4), lambda i, *s: (0, 0)),
            ],
            out_specs=pl.BlockSpec((_B, _H, _D4), lambda i, *s: (0, 0, 0)),
            scratch_shapes=[
                pltpu.VMEM((_NBUF, _CHUNK, _D), jnp.float32),
                pltpu.SemaphoreType.DMA((_NBUF,)),
                pltpu.VMEM((_H, _D), jnp.float32),
            ],
        ),
        out_shape=jax.ShapeDtypeStruct((_B, _H, _D4), jnp.float32),
    )(n, nch, cb, ci, total.reshape(1),
      episode_ctx, Wp, bp.reshape(1, _D4))

    x = jnp.concatenate([day_embed, hourly.reshape(_B, _H * _D4)], axis=-1)

    out = pl.pallas_call(
        _mlp_body,
        grid=(_NK,),
        in_specs=[
            pl.BlockSpec((_B, _KCHUNK), lambda k: (0, k)),
            pl.BlockSpec((_D, _KCHUNK), lambda k: (0, k)),
            pl.BlockSpec((1, _D), lambda k: (0, 0)),
            pl.BlockSpec((2, _D), lambda k: (0, 0)),
            pl.BlockSpec((1, 2), lambda k: (0, 0)),
        ],
        out_specs=pl.BlockSpec((_B, 2), lambda k: (0, 0)),
        out_shape=jax.ShapeDtypeStruct((_B, 2), jnp.float32),
        scratch_shapes=[pltpu.VMEM((_B, _D), jnp.float32)],
    )(x, W1, b1.reshape(1, _D), W2, b2.reshape(1, 2))

    return out


# speedup vs baseline: 1.8270x; 1.8270x over previous
import jax
import jax.numpy as jnp
from jax import lax
from jax.experimental import pallas as pl
from jax.experimental.pallas import tpu as pltpu

_B, _P, _D, _H = 16, 2048, 1024, 24
_D4 = _D // 4
_CHUNK = 256
_NBUF = 8
_XDIM = _D + _H * _D4
_KCHUNK = 1024
_NK = _XDIM // _KCHUNK


def _pool_body(s_ref, ctx_ref, wp_ref, bp_ref, out_ref, abuf, asem, acc_ref):
    def chunk_copy(b, i, slot):
        return pltpu.make_async_copy(
            ctx_ref.at[b, pl.ds(i * _CHUNK, _CHUNK), :], abuf.at[slot],
            asem.at[slot])

    h = lax.broadcasted_iota(jnp.int32, (_H, 1), 0)

    def per_sample(b, carry):
        n = s_ref[b]
        bin_size = jnp.maximum(n // _H, 1)
        start = h * bin_size
        end = jnp.minimum(start + bin_size, n)
        needed = jnp.minimum(_H * bin_size, n)
        nch = pl.cdiv(needed, _CHUNK)

        def prime(i, c):
            chunk_copy(b, i, lax.rem(i, _NBUF)).start()
            return c

        lax.fori_loop(0, jnp.minimum(nch, _NBUF), prime, 0)
        acc_ref[...] = jnp.zeros_like(acc_ref)

        def consume(i, c):
            slot = lax.rem(i, _NBUF)
            chunk_copy(b, i, slot).wait()
            pos = lax.broadcasted_iota(jnp.int32, (_H, _CHUNK), 1) + i * _CHUNK
            m = ((pos >= start) & (pos < end)).astype(jnp.bfloat16)
            acc_ref[...] += jnp.dot(m, abuf[slot].astype(jnp.bfloat16),
                                    preferred_element_type=jnp.float32)

            @pl.when(i + _NBUF < nch)
            def _():
                chunk_copy(b, i + _NBUF, slot).start()
            return c

        lax.fori_loop(0, nch, consume, 0)

        inv = 1.0 / jnp.maximum((end - start).astype(jnp.float32), 1.0)
        nonempty = (start < n).astype(jnp.float32)
        seg_mean = acc_ref[...] * (inv * nonempty)
        hourly = lax.dot_general(
            seg_mean, wp_ref[...], (((1,), (1,)), ((), ())),
            preferred_element_type=jnp.float32)
        out_ref[pl.ds(b, 1)] = (hourly + bp_ref[...] * nonempty)[None]
        return carry

    lax.fori_loop(0, _B, per_sample, 0)


def _mlp_body(x_ref, w1_ref, b1_ref, w2_ref, b2_ref, out_ref, acc_ref):
    k = pl.program_id(0)

    @pl.when(k == 0)
    def _():
        acc_ref[...] = jnp.zeros_like(acc_ref)

    acc_ref[...] += lax.dot_general(
        x_ref[...], w1_ref[...], (((1,), (1,)), ((), ())),
        preferred_element_type=jnp.float32)

    @pl.when(k == _NK - 1)
    def _():
        y = acc_ref[...] + b1_ref[...]
        y = 0.5 * y * (1.0 + lax.erf(y * 0.7071067811865476))
        out_ref[...] = lax.dot_general(
            y, w2_ref[...], (((1,), (1,)), ((), ())),
            preferred_element_type=jnp.float32) + b2_ref[...]


def kernel(day_embed, episode_ctx, n_episodes, Wp, bp, W1, b1, W2, b2):
    hourly = pl.pallas_call(
        _pool_body,
        grid_spec=pltpu.PrefetchScalarGridSpec(
            num_scalar_prefetch=1,
            grid=(1,),
            in_specs=[
                pl.BlockSpec(memory_space=pl.ANY),
                pl.BlockSpec((_D4, _D), lambda i, s: (0, 0)),
                pl.BlockSpec((1, _D4), lambda i, s: (0, 0)),
            ],
            out_specs=pl.BlockSpec((_B, _H, _D4), lambda i, s: (0, 0, 0)),
            scratch_shapes=[
                pltpu.VMEM((_NBUF, _CHUNK, _D), jnp.float32),
                pltpu.SemaphoreType.DMA((_NBUF,)),
                pltpu.VMEM((_H, _D), jnp.float32),
            ],
        ),
        out_shape=jax.ShapeDtypeStruct((_B, _H, _D4), jnp.float32),
    )(n_episodes.astype(jnp.int32), episode_ctx, Wp, bp.reshape(1, _D4))

    x = jnp.concatenate([day_embed, hourly.reshape(_B, _H * _D4)], axis=-1)

    out = pl.pallas_call(
        _mlp_body,
        grid=(_NK,),
        in_specs=[
            pl.BlockSpec((_B, _KCHUNK), lambda k: (0, k)),
            pl.BlockSpec((_D, _KCHUNK), lambda k: (0, k)),
            pl.BlockSpec((1, _D), lambda k: (0, 0)),
            pl.BlockSpec((2, _D), lambda k: (0, 0)),
            pl.BlockSpec((1, 2), lambda k: (0, 0)),
        ],
        out_specs=pl.BlockSpec((_B, 2), lambda k: (0, 0)),
        out_shape=jax.ShapeDtypeStruct((_B, 2), jnp.float32),
        scratch_shapes=[pltpu.VMEM((_B, _D), jnp.float32)],
    )(x, W1, b1.reshape(1, _D), W2, b2.reshape(1, 2))

    return out


# speedup vs baseline: 1.8673x; 1.0220x over previous
import jax
import jax.numpy as jnp
from jax import lax
from jax.experimental import pallas as pl
from jax.experimental.pallas import tpu as pltpu

_B, _P, _D, _H = 16, 2048, 1024, 24
_D4 = _D // 4
_CHUNK = 256
_NBUF = 8
_XDIM = _D + _H * _D4
_KCHUNK = 1792
_NK = _XDIM // _KCHUNK


def _pool_body(s_ref, ctx_ref, wp_ref, bp_ref, out_ref, abuf, asem, acc_ref):
    def chunk_copy(b, i, slot):
        return pltpu.make_async_copy(
            ctx_ref.at[b, pl.ds(i * _CHUNK, _CHUNK), :], abuf.at[slot],
            asem.at[slot])

    h = lax.broadcasted_iota(jnp.int32, (_H, 1), 0)

    def per_sample(b, carry):
        n = s_ref[b]
        bin_size = jnp.maximum(n // _H, 1)
        start = h * bin_size
        end = jnp.minimum(start + bin_size, n)
        needed = jnp.minimum(_H * bin_size, n)
        nch = pl.cdiv(needed, _CHUNK)

        def prime(i, c):
            chunk_copy(b, i, lax.rem(i, _NBUF)).start()
            return c

        lax.fori_loop(0, jnp.minimum(nch, _NBUF), prime, 0)
        acc_ref[...] = jnp.zeros_like(acc_ref)

        def consume(i, c):
            slot = lax.rem(i, _NBUF)
            chunk_copy(b, i, slot).wait()
            pos = lax.broadcasted_iota(jnp.int32, (_H, _CHUNK), 1) + i * _CHUNK
            m = ((pos >= start) & (pos < end)).astype(jnp.bfloat16)
            acc_ref[...] += jnp.dot(m, abuf[slot].astype(jnp.bfloat16),
                                    preferred_element_type=jnp.float32)

            @pl.when(i + _NBUF < nch)
            def _():
                chunk_copy(b, i + _NBUF, slot).start()
            return c

        lax.fori_loop(0, nch, consume, 0)

        inv = 1.0 / jnp.maximum((end - start).astype(jnp.float32), 1.0)
        nonempty = (start < n).astype(jnp.float32)
        seg_mean = acc_ref[...] * (inv * nonempty)
        hourly = lax.dot_general(
            seg_mean, wp_ref[...], (((1,), (1,)), ((), ())),
            preferred_element_type=jnp.float32)
        out_ref[pl.ds(b, 1)] = (hourly + bp_ref[...] * nonempty)[None]
        return carry

    lax.fori_loop(0, _B, per_sample, 0)


def _mlp_body(x_ref, w1_ref, b1_ref, w2_ref, b2_ref, out_ref, acc_ref):
    k = pl.program_id(0)

    @pl.when(k == 0)
    def _():
        acc_ref[...] = jnp.zeros_like(acc_ref)

    acc_ref[...] += lax.dot_general(
        x_ref[...], w1_ref[...], (((1,), (1,)), ((), ())),
        preferred_element_type=jnp.float32)

    @pl.when(k == _NK - 1)
    def _():
        y = acc_ref[...] + b1_ref[...]
        y = 0.5 * y * (1.0 + lax.erf(y * 0.7071067811865476))
        out_ref[...] = lax.dot_general(
            y, w2_ref[...], (((1,), (1,)), ((), ())),
            preferred_element_type=jnp.float32) + b2_ref[...]


def kernel(day_embed, episode_ctx, n_episodes, Wp, bp, W1, b1, W2, b2):
    hourly = pl.pallas_call(
        _pool_body,
        grid_spec=pltpu.PrefetchScalarGridSpec(
            num_scalar_prefetch=1,
            grid=(1,),
            in_specs=[
                pl.BlockSpec(memory_space=pl.ANY),
                pl.BlockSpec((_D4, _D), lambda i, s: (0, 0)),
                pl.BlockSpec((1, _D4), lambda i, s: (0, 0)),
            ],
            out_specs=pl.BlockSpec((_B, _H, _D4), lambda i, s: (0, 0, 0)),
            scratch_shapes=[
                pltpu.VMEM((_NBUF, _CHUNK, _D), jnp.float32),
                pltpu.SemaphoreType.DMA((_NBUF,)),
                pltpu.VMEM((_H, _D), jnp.float32),
            ],
        ),
        out_shape=jax.ShapeDtypeStruct((_B, _H, _D4), jnp.float32),
    )(n_episodes.astype(jnp.int32), episode_ctx, Wp, bp.reshape(1, _D4))

    x = jnp.concatenate([day_embed, hourly.reshape(_B, _H * _D4)], axis=-1)

    out = pl.pallas_call(
        _mlp_body,
        grid=(_NK,),
        in_specs=[
            pl.BlockSpec((_B, _KCHUNK), lambda k: (0, k)),
            pl.BlockSpec((_D, _KCHUNK), lambda k: (0, k)),
            pl.BlockSpec((1, _D), lambda k: (0, 0)),
            pl.BlockSpec((2, _D), lambda k: (0, 0)),
            pl.BlockSpec((1, 2), lambda k: (0, 0)),
        ],
        out_specs=pl.BlockSpec((_B, 2), lambda k: (0, 0)),
        out_shape=jax.ShapeDtypeStruct((_B, 2), jnp.float32),
        scratch_shapes=[pltpu.VMEM((_B, _D), jnp.float32)],
    )(x, W1, b1.reshape(1, _D), W2, b2.reshape(1, 2))

    return out
